# baseline (speedup 1.0000x reference)
import functools

import jax
import jax.numpy as jnp
from jax import lax
from jax.experimental import pallas as pl
from jax.experimental.pallas import tpu as pltpu
from jax.experimental.pallas import tpu_sc as plsc

N = 10000
E = 80000
R = 4
HID = 128
OUT = 64
EPS = 1e-5

NC = 2
NS = 16
B = 128
E_PAD = 81920
BPR = E_PAD // B
BPT = BPR // NS
ACC_R = 10008
NBUF = 2
CH = 624
CH_LAST = N - 15 * CH
ZR = 162
NP = 10240


def _sc_agg_body(with_cnt, *refs):
    if with_cnt:
        (x_hbm, src_hbm, dst_hbm, agg_hbm, cnt_hbm,
         sidx0, sidx1, didx0, didx1, rows0, rows1, ones_v, zc,
         acc_sp, cnt_sp,
         gsem0, gsem1, ssem0, ssem1, osem0, osem1) = refs
        osems = [osem0, osem1]
    else:
        (x_hbm, src_hbm, dst_hbm, agg_hbm,
         sidx0, sidx1, didx0, didx1, rows0, rows1,
         acc_sp,
         gsem0, gsem1, ssem0, ssem1) = refs
    sidxs = [sidx0, sidx1]
    didxs = [didx0, didx1]
    rows = [rows0, rows1]
    gsems = [gsem0, gsem1]
    ssems = [ssem0, ssem1]

    c = lax.axis_index("c")
    s = lax.axis_index("s")
    z16 = jnp.zeros((16,), jnp.float32)

    if with_cnt:
        def _zero_zc(i, _):
            zc[pl.ds(i * 16, 16)] = z16
            return 0
        lax.fori_loop(0, (NP // NS) // 16, _zero_zc, 0)
        for jj in range(B // 16):
            ones_v[pl.ds(jj * 16, 16)] = z16 + 1.0

    for p in range(NC):
        r = c * NC + p

        def _zero_rows(i, _):
            for jj in range(HID // 16):
                rows0[i, pl.ds(jj * 16, 16)] = z16
            return 0
        lax.fori_loop(0, B, _zero_rows, 0)

        @pl.when(s < NS - 1)
        def _():
            for q in range(4):
                pltpu.sync_copy(rows0, acc_sp.at[pl.ds(s * CH + q * B, B)])
            pltpu.sync_copy(rows0.at[pl.ds(0, CH - 4 * B)],
                            acc_sp.at[pl.ds(s * CH + 4 * B, CH - 4 * B)])
        @pl.when(s == NS - 1)
        def _():
            for q in range(5):
                pltpu.sync_copy(rows0, acc_sp.at[pl.ds(15 * CH + q * B, B)])
            pltpu.sync_copy(rows0.at[pl.ds(0, ACC_R - 15 * CH - 5 * B)],
                            acc_sp.at[pl.ds(15 * CH + 5 * B,
                                            ACC_R - 15 * CH - 5 * B)])
        if with_cnt:
            pltpu.sync_copy(zc, cnt_sp.at[pl.ds(s * (NP // NS), NP // NS)])
        plsc.subcore_barrier()

        base = (r * BPR + s * BPT) * B

        def _round(t, _):
            gd = [None, None]
            for b in range(NBUF):
                i = t * NBUF + b
                @pl.when(t > 0)
                def _():
                    pltpu.make_async_copy(
                        rows[b], acc_sp.at[pl.ds(0, B)], ssems[b]).wait()
                    if with_cnt:
                        pltpu.make_async_copy(
                            ones_v, cnt_sp.at[pl.ds(0, B)], osems[b]).wait()
                off = base + i * B
                pltpu.sync_copy(src_hbm.at[pl.ds(off, B)], sidxs[b])
                pltpu.sync_copy(dst_hbm.at[pl.ds(off, B)], didxs[b])
                gd[b] = pltpu.async_copy(x_hbm.at[sidxs[b]], rows[b],
                                         gsems[b])
            for b in range(NBUF):
                gd[b].wait()
                pltpu.async_copy(rows[b], acc_sp.at[didxs[b]], ssems[b],
                                 add=True)
                if with_cnt:
                    pltpu.async_copy(ones_v, cnt_sp.at[didxs[b]], osems[b],
                                     add=True)
            return 0
        lax.fori_loop(0, BPT // NBUF, _round, 0)

        for b in range(NBUF):
            pltpu.make_async_copy(rows[b], acc_sp.at[pl.ds(0, B)],
                                  ssems[b]).wait()
            if with_cnt:
                pltpu.make_async_copy(ones_v, cnt_sp.at[pl.ds(0, B)],
                                      osems[b]).wait()
        plsc.subcore_barrier()

        @pl.when(s < NS - 1)
        def _():
            pltpu.sync_copy(acc_sp.at[pl.ds(s * CH, CH)],
                            agg_hbm.at[pl.ds(r * N + s * CH, CH)])
        @pl.when(s == NS - 1)
        def _():
            pltpu.sync_copy(acc_sp.at[pl.ds(15 * CH, CH_LAST)],
                            agg_hbm.at[pl.ds(r * N + 15 * CH, CH_LAST)])
        if with_cnt:
            @pl.when(s == 0)
            def _():
                pltpu.sync_copy(cnt_sp, cnt_hbm.at[pl.ds(r * NP, NP)])
        plsc.subcore_barrier()


def _make_sc_agg(with_cnt):
    out_type = [jax.ShapeDtypeStruct((R * N, HID), jnp.float32)]
    scratch = [
        pltpu.VMEM((B,), jnp.int32),
        pltpu.VMEM((B,), jnp.int32),
        pltpu.VMEM((B,), jnp.int32),
        pltpu.VMEM((B,), jnp.int32),
        pltpu.VMEM((B, HID), jnp.float32),
        pltpu.VMEM((B, HID), jnp.float32),
    ]
    if with_cnt:
        out_type.append(jax.ShapeDtypeStruct((R * NP,), jnp.float32))
        scratch.append(pltpu.VMEM((B,), jnp.float32))
        scratch.append(pltpu.VMEM((NP // NS,), jnp.float32))
    scratch.append(pltpu.VMEM_SHARED((ACC_R, HID), jnp.float32))
    if with_cnt:
        scratch.append(pltpu.VMEM_SHARED((NP,), jnp.float32))
    scratch.extend([pltpu.SemaphoreType.DMA] * (6 if with_cnt else 4))
    return pl.kernel(
        functools.partial(_sc_agg_body, with_cnt),
        out_type=tuple(out_type),
        mesh=plsc.VectorSubcoreMesh(core_axis_name="c", subcore_axis_name="s"),
        scratch_types=tuple(scratch),
    )


def _tc_layer_body(relu, nout,
                   x_ref, agg_ref, cnt_ref, bases_ref, comp_ref, root_ref,
                   bias_ref, g_ref, b_ref, out_ref):
    x = x_ref[...]
    out = jnp.dot(x, root_ref[...], preferred_element_type=jnp.float32)
    out = out + bias_ref[...]
    inv = 1.0 / jnp.maximum(cnt_ref[...], 1.0)
    for r in range(R):
        w_r = comp_ref[r, 0] * bases_ref[0]
        for bb in range(1, R):
            w_r = w_r + comp_ref[r, bb] * bases_ref[bb]
        s_r = agg_ref[r] * inv[:, r][:, None]
        out = out + jnp.dot(s_r, w_r, preferred_element_type=jnp.float32)
    mu = jnp.mean(out, axis=1, keepdims=True)
    d = out - mu
    var = jnp.mean(d * d, axis=1, keepdims=True)
    y = d * lax.rsqrt(var + EPS) * g_ref[...] + b_ref[...]
    if relu:
        y = jnp.maximum(y, 0.0)
    out_ref[...] = y


def _tc_layer(x, agg, cnt, bases, comp, root, bias, g, b, relu):
    nout = root.shape[1]
    bn = 1000
    grid = (N // bn,)
    return pl.pallas_call(
        functools.partial(_tc_layer_body, relu, nout),
        grid=grid,
        in_specs=[
            pl.BlockSpec((bn, HID), lambda i: (i, 0)),
            pl.BlockSpec((R, bn, HID), lambda i: (0, i, 0)),
            pl.BlockSpec((bn, R), lambda i: (i, 0)),
            pl.BlockSpec((R, HID, nout), lambda i: (0, 0, 0)),
            pl.BlockSpec((R, R), lambda i: (0, 0)),
            pl.BlockSpec((HID, nout), lambda i: (0, 0)),
            pl.BlockSpec((1, nout), lambda i: (0, 0)),
            pl.BlockSpec((1, nout), lambda i: (0, 0)),
            pl.BlockSpec((1, nout), lambda i: (0, 0)),
        ],
        out_specs=pl.BlockSpec((bn, nout), lambda i: (i, 0)),
        out_shape=jax.ShapeDtypeStruct((N, nout), jnp.float32),
    )(x, agg, cnt, bases, comp, root, bias.reshape(1, nout),
      g.reshape(1, nout), b.reshape(1, nout))


_sc_agg_cnt = _make_sc_agg(True)
_sc_agg = _make_sc_agg(False)


def kernel(x_entity, edge_index_rel0, edge_index_rel1, edge_index_rel2,
           edge_index_rel3, emb, bases1, comp1, root1, bias1, ln1_g, ln1_b,
           bases2, comp2, root2, bias2, ln2_g, ln2_b):
    h = jnp.take(emb, x_entity, axis=0)
    edges = (edge_index_rel0, edge_index_rel1, edge_index_rel2,
             edge_index_rel3)

    def pad_edges(row, fill):
        parts = []
        for e in edges:
            parts.append(e[row].astype(jnp.int32))
            parts.append(jnp.full((E_PAD - E,), fill, jnp.int32))
        return jnp.concatenate(parts)

    src = pad_edges(0, 0)
    dst = pad_edges(1, N)

    agg1_flat, cnt_flat = _sc_agg_cnt(h, src, dst)
    agg1 = agg1_flat.reshape(R, N, HID)
    cnt = cnt_flat.reshape(R, NP)[:, :N]
    cnt_t = cnt.T
    h2 = _tc_layer(h, agg1, cnt_t, bases1, comp1, root1, bias1,
                   ln1_g, ln1_b, relu=True)

    (agg2_flat,) = _sc_agg(h2, src, dst)
    agg2 = agg2_flat.reshape(R, N, HID)
    out = _tc_layer(h2, agg2, cnt_t, bases2, comp2, root2, bias2,
                    ln2_g, ln2_b, relu=False)
    return out
